# baseline (speedup 1.0000x reference)
import functools
import jax
import jax.numpy as jnp
from jax import lax
from jax.experimental import pallas as pl
from jax.experimental.pallas import tpu as pltpu
from jax.experimental.pallas import tpu_sc as plsc

_N = 4096
_TOTAL = 16384
_S = 128
_COLS = 4096

_SC_ROWS = 768
_TC_ROWS = _N - _SC_ROWS
_BR = 256
_GRID = _TC_ROWS // _BR
_STAGES_PER_STEP = 9

_NW = 32
_RPW = _SC_ROWS // _NW
_RB = 4
_NB = _RPW // _RB
_OPAD = 32


def _stage_list():
    stages = []
    k = 2
    while k <= _TOTAL:
        j = k // 2
        while j >= 1:
            stages.append((k, j))
            j //= 2
        k *= 2
    return stages


_STAGES = _stage_list()


def _apply_stage(x, k, j, c_iota, r_iota):
    if j < _S:
        left = jnp.roll(x, -j, axis=1)
        right = jnp.roll(x, j, axis=1)
        islow = (c_iota & j) == 0
    else:
        jr = j // _S
        left = jnp.roll(x, -jr, axis=0)
        right = jnp.roll(x, jr, axis=0)
        islow = (r_iota & jr) == 0
    partner = jnp.where(islow, left, right)
    if k >= _TOTAL:
        up = jnp.full((_S, _S), True)
    elif k < _S:
        up = (c_iota & k) == 0
    else:
        up = (r_iota & (k // _S)) == 0
    minv = jnp.minimum(x, partner)
    maxv = jnp.maximum(x, partner)
    return jnp.where(up == islow, minv, maxv)


def _tc_body(coor_ref, lab_ref, att_ref, out_ref, sorted_out_ref,
             work_ref, sums_ref):
    g = pl.program_id(0)
    c_iota = lax.broadcasted_iota(jnp.int32, (_S, _S), 1)
    r_iota = lax.broadcasted_iota(jnp.int32, (_S, _S), 0)

    @pl.when(g == 0)
    def _():
        work_ref[...] = -att_ref[...]
        sums_ref[...] = jnp.zeros_like(sums_ref)

    n_chunks = -(-len(_STAGES) // _STAGES_PER_STEP)
    for c in range(n_chunks):
        chunk = _STAGES[c * _STAGES_PER_STEP:(c + 1) * _STAGES_PER_STEP]

        @pl.when(g == c)
        def _(chunk=chunk):
            x = work_ref[...]
            for (k, j) in chunk:
                x = _apply_stage(x, k, j, c_iota, r_iota)
            work_ref[...] = x

    s = jnp.sum(jnp.abs(coor_ref[...] - lab_ref[...]), axis=1, keepdims=True)
    nsub = _BR // _S
    for h in range(nsub):
        onehot = (c_iota[0:1, :] == g * nsub + h).astype(jnp.float32)
        sums_ref[...] += jnp.dot(
            s[_S * h:_S * (h + 1)], onehot, preferred_element_type=jnp.float32
        )

    @pl.when(g == _GRID - 1)
    def _():
        sorted_desc = -work_ref[...]
        sorted_out_ref[...] = sorted_desc
        out_ref[...] = jnp.sum(
            sorted_desc * sums_ref[...].T, dtype=jnp.float32
        ).reshape(1, 1)


def _tc_main(coor, labels, att2d):
    return pl.pallas_call(
        _tc_body,
        grid=(_GRID,),
        in_specs=[
            pl.BlockSpec((_BR, _COLS), lambda g: (g, 0)),
            pl.BlockSpec((_BR, _COLS), lambda g: (g, 0)),
            pl.BlockSpec((_S, _S), lambda g: (0, 0)),
        ],
        out_specs=[
            pl.BlockSpec((1, 1), lambda g: (0, 0)),
            pl.BlockSpec((_S, _S), lambda g: (0, 0)),
        ],
        out_shape=[
            jax.ShapeDtypeStruct((1, 1), jnp.float32),
            jax.ShapeDtypeStruct((_S, _S), jnp.float32),
        ],
        scratch_shapes=[
            pltpu.VMEM((_S, _S), jnp.float32),
            pltpu.VMEM((_S, _S), jnp.float32),
        ],
    )(coor, labels, att2d)


def _sc_body(coor_hbm, lab_hbm, out_hbm,
             a0, b0, a1, b1, o_buf, sa0, sb0, sa1, sb1):
    nc = 2
    wid = lax.axis_index("s") * nc + lax.axis_index("c")
    row0 = _TC_ROWS + wid * _RPW

    bufs = [(a0, b0, sa0, sb0), (a1, b1, sa1, sb1)]

    def issue(batch, slot):
        a, b, sa, sb = bufs[slot]
        r = row0 + batch * _RB
        ha = pltpu.async_copy(coor_hbm.at[pl.ds(r, _RB)], a, sa)
        hb = pltpu.async_copy(lab_hbm.at[pl.ds(r, _RB)], b, sb)
        return (ha, hb)

    pending = issue(0, 0)
    for bt in range(_NB):
        slot = bt % 2
        nxt = None
        if bt + 1 < _NB:
            nxt = issue(bt + 1, (bt + 1) % 2)
        ha, hb = pending
        ha.wait()
        hb.wait()
        a, b, _, _ = bufs[slot]
        for k in range(_RB):
            def col_loop(j, accs):
                c0 = j * 128
                for t in range(8):
                    av = a[k, pl.ds(c0 + 16 * t, 16)]
                    bv = b[k, pl.ds(c0 + 16 * t, 16)]
                    accs = (accs[:t % 4]
                            + (accs[t % 4] + jnp.abs(av - bv),)
                            + accs[t % 4 + 1:])
                return accs

            z = jnp.zeros((16,), jnp.float32)
            acc = lax.fori_loop(0, _COLS // 128, col_loop, (z, z, z, z))
            i = bt * _RB + k
            o_buf[pl.ds(i * 16, 16)] = (acc[0] + acc[1]) + (acc[2] + acc[3])
        pending = nxt

    pltpu.sync_copy(o_buf, out_hbm.at[pl.ds(wid * _RPW * 16, _RPW * 16)])


def _sc_row_l1(coor, labels):
    mesh = plsc.VectorSubcoreMesh(core_axis_name="c", subcore_axis_name="s")
    f = functools.partial(
        pl.kernel,
        mesh=mesh,
        out_type=jax.ShapeDtypeStruct((_SC_ROWS * 16,), jnp.float32),
        scratch_types=[
            pltpu.VMEM((_RB, _COLS), jnp.float32),
            pltpu.VMEM((_RB, _COLS), jnp.float32),
            pltpu.VMEM((_RB, _COLS), jnp.float32),
            pltpu.VMEM((_RB, _COLS), jnp.float32),
            pltpu.VMEM((_RPW * 16,), jnp.float32),
            pltpu.SemaphoreType.DMA,
            pltpu.SemaphoreType.DMA,
            pltpu.SemaphoreType.DMA,
            pltpu.SemaphoreType.DMA,
        ],
    )(_sc_body)
    return f(coor, labels)


def _combine_body(w_ref, s_ref, part_ref, out_ref):
    acc = part_ref[...]
    for q in range(_SC_ROWS // _S):
        s_col = jnp.sum(s_ref[_S * q:_S * (q + 1), :], axis=1, keepdims=True)
        acc += jnp.dot(w_ref[q:q + 1, :], s_col,
                       preferred_element_type=jnp.float32)
    out_ref[...] = acc


def _combine(w_tail, s_tail, partial):
    return pl.pallas_call(
        _combine_body,
        out_shape=jax.ShapeDtypeStruct((1, 1), jnp.float32),
    )(w_tail, s_tail, partial)


def kernel(coormeanAngles, labelsAngles, attention):
    att2d = attention.reshape(_S, _S)
    sc_flat = _sc_row_l1(coormeanAngles, labelsAngles)
    partial, sorted_desc = _tc_main(coormeanAngles, labelsAngles, att2d)
    s_tail = sc_flat.reshape(_SC_ROWS, 16)
    w_tail = sorted_desc[_TC_ROWS // _S:_N // _S, :]
    out = _combine(w_tail, s_tail, partial)
    return out[0, 0]


# speedup vs baseline: 1.4850x; 1.4850x over previous
import jax
import jax.numpy as jnp
from jax import lax
from jax.experimental import pallas as pl
from jax.experimental.pallas import tpu as pltpu

_N = 4096
_TOTAL = 16384
_S = 128
_BR = 256
_GRID = _N // _BR
_STAGES_PER_STEP = 8


def _stage_list():
    stages = []
    k = 2
    while k <= _TOTAL:
        j = k // 2
        while j >= 1:
            stages.append((k, j))
            j //= 2
        k *= 2
    return stages


_STAGES = _stage_list()


def _apply_stage(x, k, j, c_iota, r_iota):
    if j < _S:
        left = jnp.roll(x, -j, axis=1)
        right = jnp.roll(x, j, axis=1)
        islow = (c_iota & j) == 0
    else:
        jr = j // _S
        left = jnp.roll(x, -jr, axis=0)
        right = jnp.roll(x, jr, axis=0)
        islow = (r_iota & jr) == 0
    partner = jnp.where(islow, left, right)
    if k >= _TOTAL:
        up = jnp.full((_S, _S), True)
    elif k < _S:
        up = (c_iota & k) == 0
    else:
        up = (r_iota & (k // _S)) == 0
    minv = jnp.minimum(x, partner)
    maxv = jnp.maximum(x, partner)
    return jnp.where(up == islow, minv, maxv)


def _body(coor_ref, lab_ref, att_ref, out_ref, work_ref, sums_ref):
    g = pl.program_id(0)
    c_iota = lax.broadcasted_iota(jnp.int32, (_S, _S), 1)
    r_iota = lax.broadcasted_iota(jnp.int32, (_S, _S), 0)

    @pl.when(g == 0)
    def _():
        work_ref[...] = -att_ref[...]
        sums_ref[...] = jnp.zeros_like(sums_ref)
        out_ref[...] = jnp.zeros_like(out_ref)

    n_chunks = -(-len(_STAGES) // _STAGES_PER_STEP)
    for c in range(n_chunks):
        chunk = _STAGES[c * _STAGES_PER_STEP:(c + 1) * _STAGES_PER_STEP]

        @pl.when(g == c)
        def _(chunk=chunk):
            x = work_ref[...]
            for (k, j) in chunk:
                x = _apply_stage(x, k, j, c_iota, r_iota)
            work_ref[...] = x

    ones_col = jnp.ones((_N, 1), jnp.float32)
    s = jnp.dot(jnp.abs(coor_ref[...] - lab_ref[...]), ones_col,
                preferred_element_type=jnp.float32)
    nsub = _BR // _S
    for h in range(nsub):
        onehot = (c_iota[0:1, :] == g * nsub + h).astype(jnp.float32)
        sums_ref[...] += jnp.dot(
            s[_S * h:_S * (h + 1)], onehot, preferred_element_type=jnp.float32
        )

    @pl.when(g == _GRID - 1)
    def _():
        sorted_desc = -work_ref[...]
        out_ref[...] += jnp.sum(
            sorted_desc * sums_ref[...].T, dtype=jnp.float32
        ).reshape(1, 1)


def kernel(coormeanAngles, labelsAngles, attention):
    att2d = attention.reshape(_S, _S)
    out = pl.pallas_call(
        _body,
        grid=(_GRID,),
        in_specs=[
            pl.BlockSpec((_BR, _N), lambda g: (g, 0)),
            pl.BlockSpec((_BR, _N), lambda g: (g, 0)),
            pl.BlockSpec((_S, _S), lambda g: (0, 0)),
        ],
        out_specs=pl.BlockSpec((1, 1), lambda g: (0, 0)),
        out_shape=jax.ShapeDtypeStruct((1, 1), jnp.float32),
        scratch_shapes=[
            pltpu.VMEM((_S, _S), jnp.float32),
            pltpu.VMEM((_S, _S), jnp.float32),
        ],
    )(coormeanAngles, labelsAngles, att2d)
    return out[0, 0]


# speedup vs baseline: 1.5399x; 1.0370x over previous
import jax
import jax.numpy as jnp
from jax import lax
from jax.experimental import pallas as pl
from jax.experimental.pallas import tpu as pltpu

_N = 4096
_TOTAL = 16384
_S = 128
_BR = 256
_GRID = _N // _BR
_STAGES_PER_STEP = 8


def _stage_list():
    stages = []
    k = 2
    while k <= _TOTAL:
        j = k // 2
        while j >= 1:
            stages.append((k, j))
            j //= 2
        k *= 2
    return stages


_STAGES = _stage_list()


def _apply_stage(x, k, j, c_iota, r_iota):
    if j < _S:
        left = jnp.roll(x, -j, axis=1)
        right = jnp.roll(x, j, axis=1)
        islow = (c_iota & j) == 0
    else:
        jr = j // _S
        left = jnp.roll(x, -jr, axis=0)
        right = jnp.roll(x, jr, axis=0)
        islow = (r_iota & jr) == 0
    partner = jnp.where(islow, left, right)
    if k >= _TOTAL:
        up = jnp.full((_S, _S), True)
    elif k < _S:
        up = (c_iota & k) == 0
    else:
        up = (r_iota & (k // _S)) == 0
    minv = jnp.minimum(x, partner)
    maxv = jnp.maximum(x, partner)
    return jnp.where(up == islow, minv, maxv)


def _body(coor_ref, lab_ref, att_ref, out_ref, work_ref, sums_ref):
    g = pl.program_id(0)
    c_iota = lax.broadcasted_iota(jnp.int32, (_S, _S), 1)
    r_iota = lax.broadcasted_iota(jnp.int32, (_S, _S), 0)

    @pl.when(g == 0)
    def _():
        work_ref[...] = -att_ref[...]
        sums_ref[...] = jnp.zeros_like(sums_ref)
        out_ref[...] = jnp.zeros_like(out_ref)

    n_chunks = -(-len(_STAGES) // _STAGES_PER_STEP)
    for c in range(n_chunks):
        chunk = _STAGES[c * _STAGES_PER_STEP:(c + 1) * _STAGES_PER_STEP]

        @pl.when(g == c)
        def _(chunk=chunk):
            x = work_ref[...]
            for (k, j) in chunk:
                x = _apply_stage(x, k, j, c_iota, r_iota)
            work_ref[...] = x

    s = jnp.sum(jnp.abs(coor_ref[...] - lab_ref[...]), axis=1, keepdims=True)
    nsub = _BR // _S
    for h in range(nsub):
        onehot = (c_iota[0:1, :] == g * nsub + h).astype(jnp.float32)
        sums_ref[...] += jnp.dot(
            s[_S * h:_S * (h + 1)], onehot, preferred_element_type=jnp.float32
        )

    @pl.when(g == _GRID - 1)
    def _():
        sorted_desc = -work_ref[...]
        out_ref[...] += jnp.sum(
            sorted_desc * sums_ref[...].T, dtype=jnp.float32
        ).reshape(1, 1)


def kernel(coormeanAngles, labelsAngles, attention):
    att2d = attention.reshape(_S, _S)
    out = pl.pallas_call(
        _body,
        grid=(_GRID,),
        in_specs=[
            pl.BlockSpec((_BR, _N), lambda g: (g, 0)),
            pl.BlockSpec((_BR, _N), lambda g: (g, 0)),
            pl.BlockSpec((_S, _S), lambda g: (0, 0)),
        ],
        out_specs=pl.BlockSpec((1, 1), lambda g: (0, 0)),
        out_shape=jax.ShapeDtypeStruct((1, 1), jnp.float32),
        scratch_shapes=[
            pltpu.VMEM((_S, _S), jnp.float32),
            pltpu.VMEM((_S, _S), jnp.float32),
        ],
    )(coormeanAngles, labelsAngles, att2d)
    return out[0, 0]
